# Initial kernel scaffold; baseline (speedup 1.0000x reference)
import jax
import jax.numpy as jnp
from jax import lax
from jax.experimental import pallas as pl
from jax.experimental.pallas import tpu as pltpu
from jax.experimental.pallas import tpu_sc as plsc

N = 10000
E = 320000
DIN = 128
DH = 256
DOUT = 64

NC, NS = 2, 16
R = 10240
PAD = N
EROWS = 2528
EP = EROWS * 128
RT = R // NS
AROWS = EROWS // NS
DROWS = EROWS // (NC * NS)
HB = DH // 2

_MESH = plsc.VectorSubcoreMesh(core_axis_name="c", subcore_axis_name="s")


def _deg_body(dst_hbm, zeros_hbm, ones_hbm, degp_hbm, dstv, onesv, acc_sh):
    c = lax.axis_index("c")
    s = lax.axis_index("s")
    pltpu.sync_copy(zeros_hbm.at[pl.ds(s * RT, RT)], acc_sh.at[pl.ds(s * RT, RT)])
    pltpu.sync_copy(ones_hbm, onesv)
    pltpu.sync_copy(dst_hbm.at[pl.ds(c * (NS * DROWS) + s * DROWS, DROWS)], dstv)
    plsc.subcore_barrier()

    def chunk(j, carry):
        pltpu.sync_copy(onesv, acc_sh.at[dstv.at[j]], add=True)
        return carry

    lax.fori_loop(0, DROWS, chunk, 0)
    plsc.subcore_barrier()
    pltpu.sync_copy(acc_sh.at[pl.ds(s * RT, RT)], degp_hbm.at[pl.ds(c * R + s * RT, RT)])


_deg_call = pl.kernel(
    _deg_body,
    out_type=jax.ShapeDtypeStruct((2 * R, 16), jnp.float32),
    mesh=_MESH,
    scratch_types=[
        pltpu.VMEM((DROWS, 128), jnp.int32),
        pltpu.VMEM((128, 16), jnp.float32),
        pltpu.VMEM_SHARED((R, 16), jnp.float32),
    ],
)


def _agg_body(g_hbm, src_hbm, dst_hbm, out_hbm, srcv, dstv, rows, acc_sh, sem):
    c = lax.axis_index("c")
    s = lax.axis_index("s")
    pltpu.sync_copy(g_hbm.at[pl.ds(c * R + s * RT, RT)], acc_sh.at[pl.ds(s * RT, RT)])
    pltpu.sync_copy(src_hbm.at[pl.ds(c * EROWS + s * AROWS, AROWS)], srcv)
    pltpu.sync_copy(dst_hbm.at[pl.ds(s * AROWS, AROWS)], dstv)
    plsc.subcore_barrier()

    def chunk(j, carry):
        pltpu.async_copy(g_hbm.at[srcv.at[j]], rows, sem).wait()
        pltpu.sync_copy(rows, acc_sh.at[dstv.at[j]], add=True)
        return carry

    lax.fori_loop(0, AROWS, chunk, 0)
    plsc.subcore_barrier()
    pltpu.sync_copy(acc_sh.at[pl.ds(s * RT, RT)], out_hbm.at[pl.ds(c * R + s * RT, RT)])


_agg_call = pl.kernel(
    _agg_body,
    out_type=jax.ShapeDtypeStruct((2 * R, HB), jnp.float32),
    mesh=_MESH,
    scratch_types=[
        pltpu.VMEM((AROWS, 128), jnp.int32),
        pltpu.VMEM((AROWS, 128), jnp.int32),
        pltpu.VMEM((128, HB), jnp.float32),
        pltpu.VMEM_SHARED((R, HB), jnp.float32),
        pltpu.SemaphoreType.DMA,
    ],
)


def _dinv_blk(p0, p1):
    deg = p0[:, 0:1] + p1[:, 0:1] + 1.0
    return lax.rsqrt(deg)


def _tc_first_body(p0, p1, x, w, o):
    dinv = _dinv_blk(p0, p1)
    o[...] = dinv * jnp.dot(x[...], w[...], preferred_element_type=jnp.float32)


def _tc_mid_body(p0, p1, a0, a1, b, w, o):
    dinv = _dinv_blk(p0, p1)
    h = jnp.concatenate([a0[...], a1[...]], axis=1)
    z = jnp.maximum(dinv * h + b[...], 0.0)
    o[...] = dinv * jnp.dot(z, w[...], preferred_element_type=jnp.float32)


def _tc_last_body(p0, p1, a0, a1, b, w, o):
    dinv = _dinv_blk(p0, p1)
    h = dinv * jnp.concatenate([a0[...], a1[...]], axis=1) + b[...]
    o[...] = jnp.dot(h, w[...], preferred_element_type=jnp.float32)


_P0 = pl.BlockSpec((RT, 16), lambda i, j: (i, 0))
_P1 = pl.BlockSpec((RT, 16), lambda i, j: (i + NS, 0))

_tc_first = pl.pallas_call(
    _tc_first_body,
    grid=(NS, 2),
    in_specs=[
        _P0,
        _P1,
        pl.BlockSpec((RT, DIN), lambda i, j: (i, 0)),
        pl.BlockSpec((DIN, HB), lambda i, j: (0, j)),
    ],
    out_specs=pl.BlockSpec((RT, HB), lambda i, j: (j * NS + i, 0)),
    out_shape=jax.ShapeDtypeStruct((2 * R, HB), jnp.float32),
)

_tc_mid = pl.pallas_call(
    _tc_mid_body,
    grid=(NS, 2),
    in_specs=[
        _P0,
        _P1,
        pl.BlockSpec((RT, HB), lambda i, j: (i, 0)),
        pl.BlockSpec((RT, HB), lambda i, j: (i + NS, 0)),
        pl.BlockSpec((1, DH), lambda i, j: (0, 0)),
        pl.BlockSpec((DH, HB), lambda i, j: (0, j)),
    ],
    out_specs=pl.BlockSpec((RT, HB), lambda i, j: (j * NS + i, 0)),
    out_shape=jax.ShapeDtypeStruct((2 * R, HB), jnp.float32),
)

_tc_last = pl.pallas_call(
    _tc_last_body,
    grid=(NS,),
    in_specs=[
        pl.BlockSpec((RT, 16), lambda i: (i, 0)),
        pl.BlockSpec((RT, 16), lambda i: (i + NS, 0)),
        pl.BlockSpec((RT, HB), lambda i: (i, 0)),
        pl.BlockSpec((RT, HB), lambda i: (i + NS, 0)),
        pl.BlockSpec((1, DH), lambda i: (0, 0)),
        pl.BlockSpec((DH, DOUT), lambda i: (0, 0)),
    ],
    out_specs=pl.BlockSpec((RT, DOUT), lambda i: (i, 0)),
    out_shape=jax.ShapeDtypeStruct((R, DOUT), jnp.float32),
)


def kernel(x, edge_index, W1, b1, W2, b2, W3, b3, Wp):
    src = edge_index[0].astype(jnp.int32)
    dst = edge_index[1].astype(jnp.int32)
    padi = jnp.full((EP - E,), PAD, jnp.int32)
    src_p = jnp.concatenate([src, padi])
    dst_p = jnp.concatenate([dst, padi])
    src2 = jnp.concatenate([src_p, src_p + R]).reshape(2 * EROWS, 128)
    dst_r = dst_p.reshape(EROWS, 128)
    zeros16 = jnp.zeros((R, 16), jnp.float32)
    ones16 = jnp.ones((128, 16), jnp.float32)
    xp = jnp.pad(x, ((0, R - N), (0, 0)))

    degp = _deg_call(dst_r, zeros16, ones16)
    g1 = _tc_first(degp, degp, xp, W1)
    a1 = _agg_call(g1, src2, dst_r)
    g2 = _tc_mid(degp, degp, a1, a1, b1.reshape(1, DH), W2)
    a2 = _agg_call(g2, src2, dst_r)
    g3 = _tc_mid(degp, degp, a2, a2, b2.reshape(1, DH), W3)
    a3 = _agg_call(g3, src2, dst_r)
    out = _tc_last(degp, degp, a3, a3, b3.reshape(1, DH), Wp)
    return out[:N]


# speedup vs baseline: 6.6056x; 6.6056x over previous
import jax
import jax.numpy as jnp
from jax import lax
from jax.experimental import pallas as pl
from jax.experimental.pallas import tpu as pltpu
from jax.experimental.pallas import tpu_sc as plsc

N = 10000
E = 320000
DIN = 128
DH = 256
DOUT = 64

NC, NS = 2, 16
R = 10240
PAD = N
EROWS = 2560
EP = EROWS * 128
RT = R // NS
AROWS = EROWS // NS
DROWS = EROWS // (NC * NS)
HB = DH // 2

_MESH = plsc.VectorSubcoreMesh(
    core_axis_name="c", subcore_axis_name="s", num_cores=NC, num_subcores=NS
)


def _deg_body(dst_hbm, zeros_hbm, ones_hbm, degp_hbm, dstv, onesv, acc_sh):
    c = lax.axis_index("c")
    s = lax.axis_index("s")
    pltpu.sync_copy(zeros_hbm.at[pl.ds(s * RT, RT)], acc_sh.at[pl.ds(s * RT, RT)])
    pltpu.sync_copy(ones_hbm, onesv)
    plsc.subcore_barrier()

    def group(gj, carry):
        pltpu.sync_copy(dst_hbm.at[pl.ds(c * (NS * DROWS) + s * DROWS + gj * 8, 8)], dstv)

        def chunk(j, carry2):
            pltpu.sync_copy(onesv, acc_sh.at[dstv.at[j]], add=True)
            return carry2

        lax.fori_loop(0, 8, chunk, 0)
        return carry

    lax.fori_loop(0, DROWS // 8, group, 0)
    plsc.subcore_barrier()
    pltpu.sync_copy(acc_sh.at[pl.ds(s * RT, RT)], degp_hbm.at[pl.ds(c * R + s * RT, RT)])


_deg_call = pl.kernel(
    _deg_body,
    out_type=jax.ShapeDtypeStruct((2 * R, 128), jnp.float32),
    mesh=_MESH,
    scratch_types=[
        pltpu.VMEM((8, 128), jnp.int32),
        pltpu.VMEM((128, 128), jnp.float32),
        pltpu.VMEM_SHARED((R, 128), jnp.float32),
    ],
)


def _agg_body(g_hbm, src_hbm, dst_hbm, out_hbm, srcv, dstv, rows, acc_sh, sem):
    c = lax.axis_index("c")
    s = lax.axis_index("s")
    pltpu.sync_copy(g_hbm.at[pl.ds(c * R + s * RT, RT)], acc_sh.at[pl.ds(s * RT, RT)])
    plsc.subcore_barrier()

    def group(gj, carry):
        pltpu.sync_copy(src_hbm.at[pl.ds(c * EROWS + s * AROWS + gj * 8, 8)], srcv)
        pltpu.sync_copy(dst_hbm.at[pl.ds(s * AROWS + gj * 8, 8)], dstv)

        def chunk(j, carry2):
            pltpu.async_copy(g_hbm.at[srcv.at[j]], rows, sem).wait()
            pltpu.sync_copy(rows, acc_sh.at[dstv.at[j]], add=True)
            return carry2

        lax.fori_loop(0, 8, chunk, 0)
        return carry

    lax.fori_loop(0, AROWS // 8, group, 0)
    plsc.subcore_barrier()
    pltpu.sync_copy(acc_sh.at[pl.ds(s * RT, RT)], out_hbm.at[pl.ds(c * R + s * RT, RT)])


_agg_call = pl.kernel(
    _agg_body,
    out_type=jax.ShapeDtypeStruct((2 * R, HB), jnp.float32),
    mesh=_MESH,
    scratch_types=[
        pltpu.VMEM((8, 128), jnp.int32),
        pltpu.VMEM((8, 128), jnp.int32),
        pltpu.VMEM((128, HB), jnp.float32),
        pltpu.VMEM_SHARED((R, HB), jnp.float32),
        pltpu.SemaphoreType.DMA,
    ],
)


def _dinv_blk(p0, p1):
    deg = p0[:, 0:1] + p1[:, 0:1] + 1.0
    return lax.rsqrt(deg)


def _tc_first_body(p0, p1, x, w, o):
    dinv = _dinv_blk(p0, p1)
    o[...] = dinv * jnp.dot(x[...], w[...], preferred_element_type=jnp.float32)


def _tc_mid_body(p0, p1, a0, a1, b, w, o):
    dinv = _dinv_blk(p0, p1)
    h = jnp.concatenate([a0[...], a1[...]], axis=1)
    z = jnp.maximum(dinv * h + b[...], 0.0)
    o[...] = dinv * jnp.dot(z, w[...], preferred_element_type=jnp.float32)


def _tc_last_body(p0, p1, a0, a1, b, w, o):
    dinv = _dinv_blk(p0, p1)
    h = dinv * jnp.concatenate([a0[...], a1[...]], axis=1) + b[...]
    o[...] = jnp.dot(h, w[...], preferred_element_type=jnp.float32)


_P0 = pl.BlockSpec((RT, 128), lambda i, j: (i, 0))
_P1 = pl.BlockSpec((RT, 128), lambda i, j: (i + NS, 0))

_tc_first = pl.pallas_call(
    _tc_first_body,
    grid=(NS, 2),
    in_specs=[
        _P0,
        _P1,
        pl.BlockSpec((RT, DIN), lambda i, j: (i, 0)),
        pl.BlockSpec((DIN, HB), lambda i, j: (0, j)),
    ],
    out_specs=pl.BlockSpec((RT, HB), lambda i, j: (j * NS + i, 0)),
    out_shape=jax.ShapeDtypeStruct((2 * R, HB), jnp.float32),
)

_tc_mid = pl.pallas_call(
    _tc_mid_body,
    grid=(NS, 2),
    in_specs=[
        _P0,
        _P1,
        pl.BlockSpec((RT, HB), lambda i, j: (i, 0)),
        pl.BlockSpec((RT, HB), lambda i, j: (i + NS, 0)),
        pl.BlockSpec((1, DH), lambda i, j: (0, 0)),
        pl.BlockSpec((DH, HB), lambda i, j: (0, j)),
    ],
    out_specs=pl.BlockSpec((RT, HB), lambda i, j: (j * NS + i, 0)),
    out_shape=jax.ShapeDtypeStruct((2 * R, HB), jnp.float32),
)

_tc_last = pl.pallas_call(
    _tc_last_body,
    grid=(NS,),
    in_specs=[
        pl.BlockSpec((RT, 128), lambda i: (i, 0)),
        pl.BlockSpec((RT, 128), lambda i: (i + NS, 0)),
        pl.BlockSpec((RT, HB), lambda i: (i, 0)),
        pl.BlockSpec((RT, HB), lambda i: (i + NS, 0)),
        pl.BlockSpec((1, DH), lambda i: (0, 0)),
        pl.BlockSpec((DH, DOUT), lambda i: (0, 0)),
    ],
    out_specs=pl.BlockSpec((RT, DOUT), lambda i: (i, 0)),
    out_shape=jax.ShapeDtypeStruct((R, DOUT), jnp.float32),
)


def kernel(x, edge_index, W1, b1, W2, b2, W3, b3, Wp):
    src = edge_index[0].astype(jnp.int32)
    dst = edge_index[1].astype(jnp.int32)
    padi = jnp.full((EP - E,), PAD, jnp.int32)
    src_p = jnp.concatenate([src, padi])
    dst_p = jnp.concatenate([dst, padi])
    src2 = jnp.concatenate([src_p, src_p + R]).reshape(2 * EROWS, 128)
    dst_r = dst_p.reshape(EROWS, 128)
    zeros128 = jnp.zeros((R, 128), jnp.float32)
    ones128 = jnp.ones((128, 128), jnp.float32)
    xp = jnp.pad(x, ((0, R - N), (0, 0)))

    degp = _deg_call(dst_r, zeros128, ones128)
    g1 = _tc_first(degp, degp, xp, W1)
    a1 = _agg_call(g1, src2, dst_r)
    g2 = _tc_mid(degp, degp, a1, a1, b1.reshape(1, DH), W2)
    a2 = _agg_call(g2, src2, dst_r)
    g3 = _tc_mid(degp, degp, a2, a2, b2.reshape(1, DH), W3)
    a3 = _agg_call(g3, src2, dst_r)
    out = _tc_last(degp, degp, a3, a3, b3.reshape(1, DH), Wp)
    return out[:N]


# speedup vs baseline: 7.8580x; 1.1896x over previous
import jax
import jax.numpy as jnp
from jax import lax
from jax.experimental import pallas as pl
from jax.experimental.pallas import tpu as pltpu
from jax.experimental.pallas import tpu_sc as plsc

N = 10000
E = 320000
DIN = 128
DH = 256
DOUT = 64

NC, NS = 2, 16
R = 10240
PAD = N
EROWS = 2560
EP = EROWS * 128
RT = R // NS
AROWS = EROWS // NS
DROWS = EROWS // (NC * NS)
HB = DH // 2

_MESH = plsc.VectorSubcoreMesh(
    core_axis_name="c", subcore_axis_name="s", num_cores=NC, num_subcores=NS
)


def _deg_body(dst_hbm, zeros_hbm, ones_hbm, degp_hbm, dstv, onesv, acc_sh):
    c = lax.axis_index("c")
    s = lax.axis_index("s")
    pltpu.sync_copy(zeros_hbm.at[pl.ds(s * RT, RT)], acc_sh.at[pl.ds(s * RT, RT)])
    pltpu.sync_copy(ones_hbm, onesv)
    plsc.subcore_barrier()

    def group(gj, carry):
        pltpu.sync_copy(dst_hbm.at[pl.ds(c * (NS * DROWS) + s * DROWS + gj * 8, 8)], dstv)

        def chunk(j, carry2):
            pltpu.sync_copy(onesv, acc_sh.at[dstv.at[j]], add=True)
            return carry2

        lax.fori_loop(0, 8, chunk, 0)
        return carry

    lax.fori_loop(0, DROWS // 8, group, 0)
    plsc.subcore_barrier()
    pltpu.sync_copy(acc_sh.at[pl.ds(s * RT, RT)], degp_hbm.at[pl.ds(c * R + s * RT, RT)])


_deg_call = pl.kernel(
    _deg_body,
    out_type=jax.ShapeDtypeStruct((2 * R, 128), jnp.float32),
    mesh=_MESH,
    scratch_types=[
        pltpu.VMEM((8, 128), jnp.int32),
        pltpu.VMEM((128, 128), jnp.float32),
        pltpu.VMEM_SHARED((R, 128), jnp.float32),
    ],
)


AG = 16
NGROUPS = AROWS // AG


def _agg_body(g_hbm, src_hbm, dst_hbm, out_hbm, srcv, dstv, rows0, rows1,
              acc_sh, sg0, sg1, ss0, ss1):
    c = lax.axis_index("c")
    s = lax.axis_index("s")
    pltpu.sync_copy(g_hbm.at[pl.ds(c * R + s * RT, RT)], acc_sh.at[pl.ds(s * RT, RT)])
    plsc.subcore_barrier()
    rows = (rows0, rows1)
    semg = (sg0, sg1)
    sems = (ss0, ss1)

    def group(gj, carry):
        pltpu.sync_copy(src_hbm.at[pl.ds(c * EROWS + s * AROWS + gj * AG, AG)], srcv)
        pltpu.sync_copy(dst_hbm.at[pl.ds(s * AROWS + gj * AG, AG)], dstv)
        gd = [None, None]
        sd = [None, None]
        gd[0] = pltpu.async_copy(g_hbm.at[srcv.at[0]], rows[0], semg[0])
        for j in range(AG):
            b = j & 1
            nb = (j + 1) & 1
            if j + 1 < AG:
                if sd[nb] is not None:
                    sd[nb].wait()
                gd[nb] = pltpu.async_copy(g_hbm.at[srcv.at[j + 1]], rows[nb], semg[nb])
            gd[b].wait()
            sd[b] = pltpu.async_copy(rows[b], acc_sh.at[dstv.at[j]], sems[b], add=True)
        sd[0].wait()
        sd[1].wait()
        return carry

    lax.fori_loop(0, NGROUPS, group, 0)
    plsc.subcore_barrier()
    pltpu.sync_copy(acc_sh.at[pl.ds(s * RT, RT)], out_hbm.at[pl.ds(c * R + s * RT, RT)])


_agg_call = pl.kernel(
    _agg_body,
    out_type=jax.ShapeDtypeStruct((2 * R, HB), jnp.float32),
    mesh=_MESH,
    scratch_types=[
        pltpu.VMEM((AG, 128), jnp.int32),
        pltpu.VMEM((AG, 128), jnp.int32),
        pltpu.VMEM((128, HB), jnp.float32),
        pltpu.VMEM((128, HB), jnp.float32),
        pltpu.VMEM_SHARED((R, HB), jnp.float32),
        pltpu.SemaphoreType.DMA,
        pltpu.SemaphoreType.DMA,
        pltpu.SemaphoreType.DMA,
        pltpu.SemaphoreType.DMA,
    ],
)


def _dinv_blk(p0, p1):
    deg = p0[:, 0:1] + p1[:, 0:1] + 1.0
    return lax.rsqrt(deg)


def _tc_first_body(p0, p1, x, w, o):
    dinv = _dinv_blk(p0, p1)
    o[...] = dinv * jnp.dot(x[...], w[...], preferred_element_type=jnp.float32)


def _tc_mid_body(p0, p1, a0, a1, b, w, o):
    dinv = _dinv_blk(p0, p1)
    h = jnp.concatenate([a0[...], a1[...]], axis=1)
    z = jnp.maximum(dinv * h + b[...], 0.0)
    o[...] = dinv * jnp.dot(z, w[...], preferred_element_type=jnp.float32)


def _tc_last_body(p0, p1, a0, a1, b, w, o):
    dinv = _dinv_blk(p0, p1)
    h = dinv * jnp.concatenate([a0[...], a1[...]], axis=1) + b[...]
    o[...] = jnp.dot(h, w[...], preferred_element_type=jnp.float32)


_P0 = pl.BlockSpec((RT, 128), lambda i, j: (i, 0))
_P1 = pl.BlockSpec((RT, 128), lambda i, j: (i + NS, 0))

_tc_first = pl.pallas_call(
    _tc_first_body,
    grid=(NS, 2),
    in_specs=[
        _P0,
        _P1,
        pl.BlockSpec((RT, DIN), lambda i, j: (i, 0)),
        pl.BlockSpec((DIN, HB), lambda i, j: (0, j)),
    ],
    out_specs=pl.BlockSpec((RT, HB), lambda i, j: (j * NS + i, 0)),
    out_shape=jax.ShapeDtypeStruct((2 * R, HB), jnp.float32),
)

_tc_mid = pl.pallas_call(
    _tc_mid_body,
    grid=(NS, 2),
    in_specs=[
        _P0,
        _P1,
        pl.BlockSpec((RT, HB), lambda i, j: (i, 0)),
        pl.BlockSpec((RT, HB), lambda i, j: (i + NS, 0)),
        pl.BlockSpec((1, DH), lambda i, j: (0, 0)),
        pl.BlockSpec((DH, HB), lambda i, j: (0, j)),
    ],
    out_specs=pl.BlockSpec((RT, HB), lambda i, j: (j * NS + i, 0)),
    out_shape=jax.ShapeDtypeStruct((2 * R, HB), jnp.float32),
)

_tc_last = pl.pallas_call(
    _tc_last_body,
    grid=(NS,),
    in_specs=[
        pl.BlockSpec((RT, 128), lambda i: (i, 0)),
        pl.BlockSpec((RT, 128), lambda i: (i + NS, 0)),
        pl.BlockSpec((RT, HB), lambda i: (i, 0)),
        pl.BlockSpec((RT, HB), lambda i: (i + NS, 0)),
        pl.BlockSpec((1, DH), lambda i: (0, 0)),
        pl.BlockSpec((DH, DOUT), lambda i: (0, 0)),
    ],
    out_specs=pl.BlockSpec((RT, DOUT), lambda i: (i, 0)),
    out_shape=jax.ShapeDtypeStruct((R, DOUT), jnp.float32),
)


def kernel(x, edge_index, W1, b1, W2, b2, W3, b3, Wp):
    src = edge_index[0].astype(jnp.int32)
    dst = edge_index[1].astype(jnp.int32)
    padi = jnp.full((EP - E,), PAD, jnp.int32)
    src_p = jnp.concatenate([src, padi])
    dst_p = jnp.concatenate([dst, padi])
    src2 = jnp.concatenate([src_p, src_p + R]).reshape(2 * EROWS, 128)
    dst_r = dst_p.reshape(EROWS, 128)
    zeros128 = jnp.zeros((R, 128), jnp.float32)
    ones128 = jnp.ones((128, 128), jnp.float32)
    xp = jnp.pad(x, ((0, R - N), (0, 0)))

    degp = _deg_call(dst_r, zeros128, ones128)
    g1 = _tc_first(degp, degp, xp, W1)
    a1 = _agg_call(g1, src2, dst_r)
    g2 = _tc_mid(degp, degp, a1, a1, b1.reshape(1, DH), W2)
    a2 = _agg_call(g2, src2, dst_r)
    g3 = _tc_mid(degp, degp, a2, a2, b2.reshape(1, DH), W3)
    a3 = _agg_call(g3, src2, dst_r)
    out = _tc_last(degp, degp, a3, a3, b3.reshape(1, DH), Wp)
    return out[:N]


# speedup vs baseline: 7.8954x; 1.0048x over previous
import jax
import jax.numpy as jnp
from jax import lax
from jax.experimental import pallas as pl
from jax.experimental.pallas import tpu as pltpu
from jax.experimental.pallas import tpu_sc as plsc

N = 10000
E = 320000
DIN = 128
DH = 256
DOUT = 64

NC, NS = 2, 16
R = 10240
PAD = N
EROWS = 2560
EP = EROWS * 128
RT = R // NS
AROWS = EROWS // NS
DROWS = EROWS // (NC * NS)
HB = DH // 2

_MESH = plsc.VectorSubcoreMesh(
    core_axis_name="c", subcore_axis_name="s", num_cores=NC, num_subcores=NS
)


def _deg_body(dst_hbm, zeros_hbm, ones_hbm, degp_hbm, dstv, onesv, acc_sh):
    c = lax.axis_index("c")
    s = lax.axis_index("s")
    pltpu.sync_copy(zeros_hbm.at[pl.ds(s * RT, RT)], acc_sh.at[pl.ds(s * RT, RT)])
    pltpu.sync_copy(ones_hbm, onesv)
    plsc.subcore_barrier()

    def group(gj, carry):
        pltpu.sync_copy(dst_hbm.at[pl.ds(c * (NS * DROWS) + s * DROWS + gj * 8, 8)], dstv)

        def chunk(j, carry2):
            pltpu.sync_copy(onesv, acc_sh.at[dstv.at[j]], add=True)
            return carry2

        lax.fori_loop(0, 8, chunk, 0)
        return carry

    lax.fori_loop(0, DROWS // 8, group, 0)
    plsc.subcore_barrier()
    pltpu.sync_copy(acc_sh.at[pl.ds(s * RT, RT)], degp_hbm.at[pl.ds(c * R + s * RT, RT)])


_deg_call = pl.kernel(
    _deg_body,
    out_type=jax.ShapeDtypeStruct((2 * R, 128), jnp.float32),
    mesh=_MESH,
    scratch_types=[
        pltpu.VMEM((8, 128), jnp.int32),
        pltpu.VMEM((128, 128), jnp.float32),
        pltpu.VMEM_SHARED((R, 128), jnp.float32),
    ],
)


AG = 16
NGROUPS = AROWS // AG


def _agg_body(g_hbm, src_hbm, dst_hbm, out_hbm, srcv, dstv, rows0, rows1,
              acc_sh, sg0, sg1, ss0, ss1):
    c = lax.axis_index("c")
    s = lax.axis_index("s")
    pltpu.sync_copy(g_hbm.at[pl.ds(c * R + s * RT, RT)], acc_sh.at[pl.ds(s * RT, RT)])
    plsc.subcore_barrier()
    rows = (rows0, rows1)
    semg = (sg0, sg1)
    sems = (ss0, ss1)

    def group(gj, carry):
        pltpu.sync_copy(src_hbm.at[pl.ds(c * EROWS + s * AROWS + gj * AG, AG)], srcv)
        pltpu.sync_copy(dst_hbm.at[pl.ds(s * AROWS + gj * AG, AG)], dstv)
        def gat(j, b):
            d0 = pltpu.async_copy(
                g_hbm.at[srcv.at[j, pl.ds(0, 64)]], rows[b].at[pl.ds(0, 64)], semg[b])
            d1 = pltpu.async_copy(
                g_hbm.at[srcv.at[j, pl.ds(64, 64)]], rows[b].at[pl.ds(64, 64)], semg[b])
            return (d0, d1)

        gd = [None, None]
        sd = [None, None]
        gd[0] = gat(0, 0)
        for j in range(AG):
            b = j & 1
            nb = (j + 1) & 1
            if j + 1 < AG:
                if sd[nb] is not None:
                    sd[nb].wait()
                gd[nb] = gat(j + 1, nb)
            gd[b][0].wait()
            gd[b][1].wait()
            sd[b] = pltpu.async_copy(rows[b], acc_sh.at[dstv.at[j]], sems[b], add=True)
        sd[0].wait()
        sd[1].wait()
        return carry

    lax.fori_loop(0, NGROUPS, group, 0)
    plsc.subcore_barrier()
    pltpu.sync_copy(acc_sh.at[pl.ds(s * RT, RT)], out_hbm.at[pl.ds(c * R + s * RT, RT)])


_agg_call = pl.kernel(
    _agg_body,
    out_type=jax.ShapeDtypeStruct((2 * R, HB), jnp.float32),
    mesh=_MESH,
    scratch_types=[
        pltpu.VMEM((AG, 128), jnp.int32),
        pltpu.VMEM((AG, 128), jnp.int32),
        pltpu.VMEM((128, HB), jnp.float32),
        pltpu.VMEM((128, HB), jnp.float32),
        pltpu.VMEM_SHARED((R, HB), jnp.float32),
        pltpu.SemaphoreType.DMA,
        pltpu.SemaphoreType.DMA,
        pltpu.SemaphoreType.DMA,
        pltpu.SemaphoreType.DMA,
    ],
)


def _dinv_blk(p0, p1):
    deg = p0[:, 0:1] + p1[:, 0:1] + 1.0
    return lax.rsqrt(deg)


def _tc_first_body(p0, p1, x, w, o):
    dinv = _dinv_blk(p0, p1)
    o[...] = dinv * jnp.dot(x[...], w[...], preferred_element_type=jnp.float32)


def _tc_mid_body(p0, p1, a0, a1, b, w, o):
    dinv = _dinv_blk(p0, p1)
    h = jnp.concatenate([a0[...], a1[...]], axis=1)
    z = jnp.maximum(dinv * h + b[...], 0.0)
    o[...] = dinv * jnp.dot(z, w[...], preferred_element_type=jnp.float32)


def _tc_last_body(p0, p1, a0, a1, b, w, o):
    dinv = _dinv_blk(p0, p1)
    h = dinv * jnp.concatenate([a0[...], a1[...]], axis=1) + b[...]
    o[...] = jnp.dot(h, w[...], preferred_element_type=jnp.float32)


_P0 = pl.BlockSpec((RT, 128), lambda i, j: (i, 0))
_P1 = pl.BlockSpec((RT, 128), lambda i, j: (i + NS, 0))

_tc_first = pl.pallas_call(
    _tc_first_body,
    grid=(NS, 2),
    in_specs=[
        _P0,
        _P1,
        pl.BlockSpec((RT, DIN), lambda i, j: (i, 0)),
        pl.BlockSpec((DIN, HB), lambda i, j: (0, j)),
    ],
    out_specs=pl.BlockSpec((RT, HB), lambda i, j: (j * NS + i, 0)),
    out_shape=jax.ShapeDtypeStruct((2 * R, HB), jnp.float32),
)

_tc_mid = pl.pallas_call(
    _tc_mid_body,
    grid=(NS, 2),
    in_specs=[
        _P0,
        _P1,
        pl.BlockSpec((RT, HB), lambda i, j: (i, 0)),
        pl.BlockSpec((RT, HB), lambda i, j: (i + NS, 0)),
        pl.BlockSpec((1, DH), lambda i, j: (0, 0)),
        pl.BlockSpec((DH, HB), lambda i, j: (0, j)),
    ],
    out_specs=pl.BlockSpec((RT, HB), lambda i, j: (j * NS + i, 0)),
    out_shape=jax.ShapeDtypeStruct((2 * R, HB), jnp.float32),
)

_tc_last = pl.pallas_call(
    _tc_last_body,
    grid=(NS,),
    in_specs=[
        pl.BlockSpec((RT, 128), lambda i: (i, 0)),
        pl.BlockSpec((RT, 128), lambda i: (i + NS, 0)),
        pl.BlockSpec((RT, HB), lambda i: (i, 0)),
        pl.BlockSpec((RT, HB), lambda i: (i + NS, 0)),
        pl.BlockSpec((1, DH), lambda i: (0, 0)),
        pl.BlockSpec((DH, DOUT), lambda i: (0, 0)),
    ],
    out_specs=pl.BlockSpec((RT, DOUT), lambda i: (i, 0)),
    out_shape=jax.ShapeDtypeStruct((R, DOUT), jnp.float32),
)


def kernel(x, edge_index, W1, b1, W2, b2, W3, b3, Wp):
    src = edge_index[0].astype(jnp.int32)
    dst = edge_index[1].astype(jnp.int32)
    padi = jnp.full((EP - E,), PAD, jnp.int32)
    src_p = jnp.concatenate([src, padi])
    dst_p = jnp.concatenate([dst, padi])
    src2 = jnp.concatenate([src_p, src_p + R]).reshape(2 * EROWS, 128)
    dst_r = dst_p.reshape(EROWS, 128)
    zeros128 = jnp.zeros((R, 128), jnp.float32)
    ones128 = jnp.ones((128, 128), jnp.float32)
    xp = jnp.pad(x, ((0, R - N), (0, 0)))

    degp = _deg_call(dst_r, zeros128, ones128)
    g1 = _tc_first(degp, degp, xp, W1)
    a1 = _agg_call(g1, src2, dst_r)
    g2 = _tc_mid(degp, degp, a1, a1, b1.reshape(1, DH), W2)
    a2 = _agg_call(g2, src2, dst_r)
    g3 = _tc_mid(degp, degp, a2, a2, b2.reshape(1, DH), W3)
    a3 = _agg_call(g3, src2, dst_r)
    out = _tc_last(degp, degp, a3, a3, b3.reshape(1, DH), Wp)
    return out[:N]


# speedup vs baseline: 8.6500x; 1.0956x over previous
import jax
import jax.numpy as jnp
from jax import lax
from jax.experimental import pallas as pl
from jax.experimental.pallas import tpu as pltpu
from jax.experimental.pallas import tpu_sc as plsc

N = 10000
E = 320000
DIN = 128
DH = 256
DOUT = 64

NC, NS = 2, 16
R = 10112
PAD = N
EROWS = 2560
EP = EROWS * 128
RT = R // NS
AROWS = EROWS // NS
DROWS = EROWS // (NC * NS)
HB = DH // 2

_MESH = plsc.VectorSubcoreMesh(
    core_axis_name="c", subcore_axis_name="s", num_cores=NC, num_subcores=NS
)


def _deg_body(dst_hbm, zeros_hbm, ones_hbm, degp_hbm, dstv, onesv, acc_sh):
    c = lax.axis_index("c")
    s = lax.axis_index("s")
    pltpu.sync_copy(zeros_hbm.at[pl.ds(s * RT, RT)], acc_sh.at[pl.ds(s * RT, RT)])
    pltpu.sync_copy(ones_hbm, onesv)
    plsc.subcore_barrier()

    def group(gj, carry):
        pltpu.sync_copy(dst_hbm.at[pl.ds(c * (NS * DROWS) + s * DROWS + gj * 8, 8)], dstv)

        def chunk(j, carry2):
            pltpu.sync_copy(onesv, acc_sh.at[dstv.at[j]], add=True)
            return carry2

        lax.fori_loop(0, 8, chunk, 0)
        return carry

    lax.fori_loop(0, DROWS // 8, group, 0)
    plsc.subcore_barrier()
    pltpu.sync_copy(acc_sh.at[pl.ds(s * RT, RT)], degp_hbm.at[pl.ds(c * R + s * RT, RT)])


_deg_call = pl.kernel(
    _deg_body,
    out_type=jax.ShapeDtypeStruct((2 * R, 128), jnp.float32),
    mesh=_MESH,
    scratch_types=[
        pltpu.VMEM((8, 128), jnp.int32),
        pltpu.VMEM((128, 128), jnp.float32),
        pltpu.VMEM_SHARED((R, 128), jnp.float32),
    ],
)


AG = 8
NSG = AROWS // (2 * AG)


def _agg_body(g_hbm, src_hbm, dst_hbm, out_hbm, srcvA, dstvA, srcvB, dstvB,
              rows0, rows1, acc_sh, sg0, sg1, ss0, ss1, si):
    c = lax.axis_index("c")
    s = lax.axis_index("s")
    rows = (rows0, rows1)
    semg = (sg0, sg1)
    sems = (ss0, ss1)
    slabs = ((srcvA, dstvA), (srcvB, dstvB))
    sbase = c * EROWS + s * AROWS
    dbase = s * AROWS

    def load_slabs(p, row0):
        pltpu.async_copy(src_hbm.at[pl.ds(sbase + row0, AG)], slabs[p][0], si)
        pltpu.async_copy(dst_hbm.at[pl.ds(dbase + row0, AG)], slabs[p][1], si)

    def wait_slabs(p):
        pltpu.make_async_copy(src_hbm.at[pl.ds(sbase, AG)], slabs[p][0], si).wait()
        pltpu.make_async_copy(dst_hbm.at[pl.ds(dbase, AG)], slabs[p][1], si).wait()

    load_slabs(0, 0)
    pltpu.sync_copy(g_hbm.at[pl.ds(c * R + s * RT, RT)], acc_sh.at[pl.ds(s * RT, RT)])
    plsc.subcore_barrier()

    def sgroup(gj, carry):
        wait_slabs(0)
        gd = [None, None]
        sd = [None, None]
        gd[0] = pltpu.async_copy(g_hbm.at[slabs[0][0].at[0]], rows[0], semg[0])
        for jj in range(2 * AG):
            p, j = divmod(jj, AG)
            b = jj & 1
            nb = (jj + 1) & 1
            if jj == 0:
                load_slabs(1, gj * 2 * AG + AG)
            if jj == AG:
                wait_slabs(1)
            if jj == AG + 2:
                @pl.when(gj < NSG - 1)
                def _():
                    load_slabs(0, (gj + 1) * 2 * AG)
            if jj + 1 < 2 * AG:
                np_, nj = divmod(jj + 1, AG)
                if sd[nb] is not None:
                    sd[nb].wait()
                gd[nb] = pltpu.async_copy(
                    g_hbm.at[slabs[np_][0].at[nj]], rows[nb], semg[nb])
            gd[b].wait()
            sd[b] = pltpu.async_copy(rows[b], acc_sh.at[slabs[p][1].at[j]], sems[b], add=True)
        sd[0].wait()
        sd[1].wait()
        return carry

    lax.fori_loop(0, NSG, sgroup, 0)
    plsc.subcore_barrier()
    pltpu.sync_copy(acc_sh.at[pl.ds(s * RT, RT)], out_hbm.at[pl.ds(c * R + s * RT, RT)])


_agg_call = pl.kernel(
    _agg_body,
    out_type=jax.ShapeDtypeStruct((2 * R, HB), jnp.float32),
    mesh=_MESH,
    scratch_types=[
        pltpu.VMEM((AG, 128), jnp.int32),
        pltpu.VMEM((AG, 128), jnp.int32),
        pltpu.VMEM((AG, 128), jnp.int32),
        pltpu.VMEM((AG, 128), jnp.int32),
        pltpu.VMEM((128, HB), jnp.float32),
        pltpu.VMEM((128, HB), jnp.float32),
        pltpu.VMEM_SHARED((R, HB), jnp.float32),
        pltpu.SemaphoreType.DMA,
        pltpu.SemaphoreType.DMA,
        pltpu.SemaphoreType.DMA,
        pltpu.SemaphoreType.DMA,
        pltpu.SemaphoreType.DMA,
    ],
)


def _dinv_blk(p0, p1):
    deg = p0[:, 0:1] + p1[:, 0:1] + 1.0
    return lax.rsqrt(deg)


def _tc_first_body(p0, p1, x, w, o):
    dinv = _dinv_blk(p0, p1)
    o[...] = dinv * jnp.dot(x[...], w[...], preferred_element_type=jnp.float32)


def _tc_mid_body(p0, p1, a0, a1, b, w, o):
    dinv = _dinv_blk(p0, p1)
    h = jnp.concatenate([a0[...], a1[...]], axis=1)
    z = jnp.maximum(dinv * h + b[...], 0.0)
    o[...] = dinv * jnp.dot(z, w[...], preferred_element_type=jnp.float32)


def _tc_last_body(p0, p1, a0, a1, b, w, o):
    dinv = _dinv_blk(p0, p1)
    h = dinv * jnp.concatenate([a0[...], a1[...]], axis=1) + b[...]
    o[...] = jnp.dot(h, w[...], preferred_element_type=jnp.float32)


_P0 = pl.BlockSpec((RT, 128), lambda i, j: (i, 0))
_P1 = pl.BlockSpec((RT, 128), lambda i, j: (i + NS, 0))

_tc_first = pl.pallas_call(
    _tc_first_body,
    grid=(NS, 2),
    in_specs=[
        _P0,
        _P1,
        pl.BlockSpec((RT, DIN), lambda i, j: (i, 0)),
        pl.BlockSpec((DIN, HB), lambda i, j: (0, j)),
    ],
    out_specs=pl.BlockSpec((RT, HB), lambda i, j: (j * NS + i, 0)),
    out_shape=jax.ShapeDtypeStruct((2 * R, HB), jnp.float32),
)

_tc_mid = pl.pallas_call(
    _tc_mid_body,
    grid=(NS, 2),
    in_specs=[
        _P0,
        _P1,
        pl.BlockSpec((RT, HB), lambda i, j: (i, 0)),
        pl.BlockSpec((RT, HB), lambda i, j: (i + NS, 0)),
        pl.BlockSpec((1, DH), lambda i, j: (0, 0)),
        pl.BlockSpec((DH, HB), lambda i, j: (0, j)),
    ],
    out_specs=pl.BlockSpec((RT, HB), lambda i, j: (j * NS + i, 0)),
    out_shape=jax.ShapeDtypeStruct((2 * R, HB), jnp.float32),
)

_tc_last = pl.pallas_call(
    _tc_last_body,
    grid=(NS,),
    in_specs=[
        pl.BlockSpec((RT, 128), lambda i: (i, 0)),
        pl.BlockSpec((RT, 128), lambda i: (i + NS, 0)),
        pl.BlockSpec((RT, HB), lambda i: (i, 0)),
        pl.BlockSpec((RT, HB), lambda i: (i + NS, 0)),
        pl.BlockSpec((1, DH), lambda i: (0, 0)),
        pl.BlockSpec((DH, DOUT), lambda i: (0, 0)),
    ],
    out_specs=pl.BlockSpec((RT, DOUT), lambda i: (i, 0)),
    out_shape=jax.ShapeDtypeStruct((R, DOUT), jnp.float32),
)


def kernel(x, edge_index, W1, b1, W2, b2, W3, b3, Wp):
    src = edge_index[0].astype(jnp.int32)
    dst = edge_index[1].astype(jnp.int32)
    padi = jnp.full((EP - E,), PAD, jnp.int32)
    src_p = jnp.concatenate([src, padi])
    dst_p = jnp.concatenate([dst, padi])
    src2 = jnp.concatenate([src_p, src_p + R]).reshape(2 * EROWS, 128)
    dst_r = dst_p.reshape(EROWS, 128)
    zeros128 = jnp.zeros((R, 128), jnp.float32)
    ones128 = jnp.ones((128, 128), jnp.float32)
    xp = jnp.pad(x, ((0, R - N), (0, 0)))

    degp = _deg_call(dst_r, zeros128, ones128)
    g1 = _tc_first(degp, degp, xp, W1)
    a1 = _agg_call(g1, src2, dst_r)
    g2 = _tc_mid(degp, degp, a1, a1, b1.reshape(1, DH), W2)
    a2 = _agg_call(g2, src2, dst_r)
    g3 = _tc_mid(degp, degp, a2, a2, b2.reshape(1, DH), W3)
    a3 = _agg_call(g3, src2, dst_r)
    out = _tc_last(degp, degp, a3, a3, b3.reshape(1, DH), Wp)
    return out[:N]
